# baseline (speedup 1.0000x reference)
import functools

import jax
import jax.numpy as jnp
from jax import lax
from jax.experimental import pallas as pl
from jax.experimental.pallas import tpu as pltpu
from jax.experimental.pallas import tpu_sc as plsc

D = 128
NW = 32
CW0 = 128
NBUF = 4
NGRP = 4


def _make_embed(B: int, S: int, N: int):
    BW = B * S
    BN = B * N
    RPW = B // NW
    wpw = RPW * S
    npw = RPW * N
    CW1 = S - CW0
    G = RPW // NGRP
    GID = NGRP * N
    assert RPW % NGRP == 0 and 0 < CW1 <= 128 and GID % 8 == 0

    mesh = plsc.VectorSubcoreMesh(core_axis_name="c", subcore_axis_name="s")

    @functools.partial(
        pl.kernel,
        mesh=mesh,
        compiler_params=pltpu.CompilerParams(use_tc_tiling_on_sc=True),
        out_type=(
            jax.ShapeDtypeStruct((B, S, D), jnp.float32),
            jax.ShapeDtypeStruct((B, N, D), jnp.float32),
        ),
        scratch_types=[
            pltpu.VMEM((RPW, S), jnp.int32),
            pltpu.VMEM((RPW, N), jnp.int32),
            pltpu.VMEM((2, CW0 + CW1, D), jnp.float32),
            pltpu.VMEM((2, NGRP, N, D), jnp.float32),
            pltpu.SemaphoreType.DMA,
            pltpu.SemaphoreType.DMA,
        ]
        + [pltpu.SemaphoreType.DMA] * (2 * NBUF),
    )
    def embed(x_hbm, nid_hbm, ww_hbm, wg_hbm, out_w, out_n,
              idxw_2d, idxn_2d, bufs_v, bufn_v, isem0, isem1, *sems):
        gsems = sems[:NBUF]
        osems = sems[NBUF:]
        wid = lax.axis_index("s") * 2 + lax.axis_index("c")
        rbase = wid * RPW

        widx_cp = pltpu.make_async_copy(
            x_hbm.at[pl.ds(wid * RPW, RPW), :], idxw_2d, isem0)
        nidx_cp = pltpu.make_async_copy(
            nid_hbm.at[pl.ds(wid * RPW, RPW), :], idxn_2d, isem1)
        widx_cp.start()
        nidx_cp.start()

        def wsize(b):
            return CW0 if b % 2 == 0 else CW1

        def wbuf(b):
            return bufs_v.at[b // 2, pl.ds((b % 2) * CW0, wsize(b))]

        def wgather(j, b):
            row = j // 2
            idx = idxw_2d.at[row, pl.ds((b % 2) * CW0, wsize(b))]
            return pltpu.make_async_copy(ww_hbm.at[idx], wbuf(b), gsems[b])

        def wocopy(j, b):
            row = j // 2
            dst = out_w.at[rbase + row, pl.ds((b % 2) * CW0, wsize(b))]
            return pltpu.make_async_copy(wbuf(b), dst, osems[b])

        def word_ring(nchunks):
            for b in range(NBUF):
                wgather(b, b).start()

            def body(i, carry):
                j0 = i * NBUF
                for b in range(NBUF):
                    wgather(j0 + b, b).wait()
                    wocopy(j0 + b, b).start()
                for b in range(NBUF):
                    wocopy(j0 + b, b).wait()

                    @pl.when(j0 + b + NBUF < nchunks)
                    def _():
                        wgather(j0 + b + NBUF, b).start()
                return carry

            lax.fori_loop(0, nchunks // NBUF, body, 0)

        def ngathers(g, b):
            descs = []
            for r in range(NGRP):
                idx = idxn_2d.at[g * NGRP + r]
                buf = bufn_v.at[b, r]
                descs.append(
                    pltpu.make_async_copy(wg_hbm.at[idx], buf, gsems[b]))
            return descs

        def nocopy(g, b):
            dst = out_n.at[pl.ds(rbase + g * NGRP, NGRP)]
            return pltpu.make_async_copy(bufn_v.at[b], dst, osems[b])

        def node_ring(ngroups):
            for b in range(2):
                for d in ngathers(b, b):
                    d.start()

            def body(i, carry):
                g0 = i * 2
                for b in range(2):
                    for d in ngathers(g0 + b, b):
                        d.wait()
                    nocopy(g0 + b, b).start()
                for b in range(2):
                    nocopy(g0 + b, b).wait()

                    @pl.when(g0 + b + 2 < ngroups)
                    def _():
                        for d in ngathers(g0 + b + 2, b):
                            d.start()
                return carry

            lax.fori_loop(0, ngroups // 2, body, 0)

        widx_cp.wait()
        word_ring(2 * RPW)
        nidx_cp.wait()
        node_ring(G)

    return embed


def kernel(x, node_ids, W_word, W_graph):
    B, S = x.shape
    _, N = node_ids.shape
    return _make_embed(B, S, N)(x, node_ids, W_word, W_graph)


# speedup vs baseline: 1.0109x; 1.0109x over previous
import functools

import jax
import jax.numpy as jnp
from jax import lax
from jax.experimental import pallas as pl
from jax.experimental.pallas import tpu as pltpu
from jax.experimental.pallas import tpu_sc as plsc

D = 128
NW = 32
CW0 = 128
NBUF = 6
NGRP = 2


def _make_embed(B: int, S: int, N: int):
    BW = B * S
    BN = B * N
    RPW = B // NW
    wpw = RPW * S
    npw = RPW * N
    CW1 = S - CW0
    G = RPW // NGRP
    GID = NGRP * N
    assert RPW % NGRP == 0 and 0 < CW1 <= 128 and GID <= 128

    mesh = plsc.VectorSubcoreMesh(core_axis_name="c", subcore_axis_name="s")

    @functools.partial(
        pl.kernel,
        mesh=mesh,
        out_type=(
            jax.ShapeDtypeStruct((B, S, D), jnp.float32),
            jax.ShapeDtypeStruct((B, N, D), jnp.float32),
        ),
        scratch_types=[
            pltpu.VMEM((RPW, S), jnp.int32),
            pltpu.VMEM((RPW, N), jnp.int32),
            pltpu.VMEM((NBUF // 2, CW0 + CW1, D), jnp.float32),
            pltpu.VMEM((2, NGRP, N, D), jnp.float32),
            pltpu.SemaphoreType.DMA,
            pltpu.SemaphoreType.DMA,
        ]
        + [pltpu.SemaphoreType.DMA] * (2 * NBUF),
    )
    def embed(x_hbm, nid_hbm, ww_hbm, wg_hbm, out_w, out_n,
              idxw_2d, idxn_2d, bufs_v, bufn_v, isem0, isem1, *sems):
        gsems = sems[:NBUF]
        osems = sems[NBUF:]
        wid = lax.axis_index("s") * 2 + lax.axis_index("c")
        rbase = wid * RPW

        widx_cp = pltpu.make_async_copy(
            x_hbm.at[pl.ds(wid * RPW, RPW), :], idxw_2d, isem0)
        nidx_cp = pltpu.make_async_copy(
            nid_hbm.at[pl.ds(wid * RPW, RPW), :], idxn_2d, isem1)
        widx_cp.start()
        nidx_cp.start()

        def wsize(b):
            return CW0 if b % 2 == 0 else CW1

        def wbuf(b):
            return bufs_v.at[b // 2, pl.ds((b % 2) * CW0, wsize(b))]

        def wgather(j, b):
            row = j // 2
            idx = idxw_2d.at[row, pl.ds((b % 2) * CW0, wsize(b))]
            return pltpu.make_async_copy(ww_hbm.at[idx], wbuf(b), gsems[b])

        def wocopy(j, b):
            row = j // 2
            dst = out_w.at[rbase + row, pl.ds((b % 2) * CW0, wsize(b))]
            return pltpu.make_async_copy(wbuf(b), dst, osems[b])

        def word_ring(nchunks):
            for b in range(min(NBUF, nchunks)):
                wgather(b, b).start()

            def body(i, carry):
                j0 = i * NBUF
                for b in range(NBUF):
                    j = j0 + b

                    @pl.when(j < nchunks)
                    def _():
                        wgather(j, b).wait()
                        wocopy(j, b).start()
                for b in range(NBUF):
                    j = j0 + b

                    @pl.when(j < nchunks)
                    def _():
                        wocopy(j, b).wait()

                        @pl.when(j + NBUF < nchunks)
                        def _():
                            wgather(j + NBUF, b).start()
                return carry

            lax.fori_loop(0, (nchunks + NBUF - 1) // NBUF, body, 0)

        def ngathers(g, b):
            return [
                pltpu.make_async_copy(
                    wg_hbm.at[idxn_2d.at[g * NGRP + r]],
                    bufn_v.at[b, r], gsems[b])
                for r in range(NGRP)
            ]

        def nocopy(g, b):
            dst = out_n.at[pl.ds(rbase + g * NGRP, NGRP)]
            return pltpu.make_async_copy(bufn_v.at[b], dst, osems[b])

        def node_ring(ngroups):
            for b in range(2):
                for d in ngathers(b, b):
                    d.start()

            def body(i, carry):
                g0 = i * 2
                for b in range(2):
                    for d in ngathers(g0 + b, b):
                        d.wait()
                    nocopy(g0 + b, b).start()
                for b in range(2):
                    nocopy(g0 + b, b).wait()

                    @pl.when(g0 + b + 2 < ngroups)
                    def _():
                        for d in ngathers(g0 + b + 2, b):
                            d.start()
                return carry

            lax.fori_loop(0, ngroups // 2, body, 0)

        widx_cp.wait()
        word_ring(2 * RPW)
        nidx_cp.wait()
        node_ring(G)

    return embed


def kernel(x, node_ids, W_word, W_graph):
    B, S = x.shape
    _, N = node_ids.shape
    return _make_embed(B, S, N)(x, node_ids, W_word, W_graph)


# speedup vs baseline: 1.0395x; 1.0283x over previous
import functools

import jax
import jax.numpy as jnp
from jax import lax
from jax.experimental import pallas as pl
from jax.experimental.pallas import tpu as pltpu
from jax.experimental.pallas import tpu_sc as plsc

D = 128
NW = 32
CW0 = 128
NBUF = 4
NGRP = 2


def _make_embed(B: int, S: int, N: int):
    BW = B * S
    BN = B * N
    RPW = B // NW
    wpw = RPW * S
    npw = RPW * N
    CW1 = S - CW0
    G = RPW // NGRP
    GID = NGRP * N
    assert RPW % NGRP == 0 and 0 < CW1 <= 128 and GID <= 128

    mesh = plsc.VectorSubcoreMesh(core_axis_name="c", subcore_axis_name="s")

    @functools.partial(
        pl.kernel,
        mesh=mesh,
        out_type=(
            jax.ShapeDtypeStruct((B, S, D), jnp.float32),
            jax.ShapeDtypeStruct((B, N, D), jnp.float32),
        ),
        scratch_types=[
            pltpu.VMEM((RPW, S), jnp.int32),
            pltpu.VMEM((RPW, N), jnp.int32),
            pltpu.VMEM((NBUF // 2, CW0 + CW1, D), jnp.float32),
            pltpu.VMEM((2, NGRP, N, D), jnp.float32),
            pltpu.SemaphoreType.DMA,
            pltpu.SemaphoreType.DMA,
        ]
        + [pltpu.SemaphoreType.DMA] * 12,
    )
    def embed(x_hbm, nid_hbm, ww_hbm, wg_hbm, out_w, out_n,
              idxw_2d, idxn_2d, bufs_v, bufn_v, isem0, isem1, *sems):
        gsems = sems[0:4]
        osems = sems[4:8]
        nsems = sems[8:10]
        msems = sems[10:12]
        wid = lax.axis_index("s") * 2 + lax.axis_index("c")
        rbase = wid * RPW

        widx_cp = pltpu.make_async_copy(
            x_hbm.at[pl.ds(wid * RPW, RPW), :], idxw_2d, isem0)
        nidx_cp = pltpu.make_async_copy(
            nid_hbm.at[pl.ds(wid * RPW, RPW), :], idxn_2d, isem1)
        widx_cp.start()
        nidx_cp.start()

        def wsize(b):
            return CW0 if b % 2 == 0 else CW1

        def wbuf(b):
            return bufs_v.at[b // 2, pl.ds((b % 2) * CW0, wsize(b))]

        def wgather(j, b):
            row = j // 2
            idx = idxw_2d.at[row, pl.ds((b % 2) * CW0, wsize(b))]
            return pltpu.make_async_copy(ww_hbm.at[idx], wbuf(b), gsems[b])

        def wocopy(j, b):
            row = j // 2
            dst = out_w.at[rbase + row, pl.ds((b % 2) * CW0, wsize(b))]
            return pltpu.make_async_copy(wbuf(b), dst, osems[b])

        def ngathers(g, b):
            return [
                pltpu.make_async_copy(
                    wg_hbm.at[idxn_2d.at[g * NGRP + r]],
                    bufn_v.at[b, r], nsems[b])
                for r in range(NGRP)
            ]

        def nocopy(g, b):
            dst = out_n.at[pl.ds(rbase + g * NGRP, NGRP)]
            return pltpu.make_async_copy(bufn_v.at[b], dst, msems[b])

        nchunks = 2 * RPW
        ngroups = G

        widx_cp.wait()
        for b in range(4):
            wgather(b, b).start()
        nidx_cp.wait()
        for h in range(2):
            for d in ngathers(h, h):
                d.start()

        def body(i, carry):
            for h in range(2):
                j0 = i * 8 + 4 * h
                g = 2 * i + h
                for b in range(4):
                    wgather(j0 + b, b).wait()
                    wocopy(j0 + b, b).start()
                for d in ngathers(g, h):
                    d.wait()
                nocopy(g, h).start()
                for b in range(4):
                    wocopy(j0 + b, b).wait()

                    @pl.when(j0 + b + 4 < nchunks)
                    def _():
                        wgather(j0 + b + 4, b).start()
                nocopy(g, h).wait()

                @pl.when(g + 2 < ngroups)
                def _():
                    for d in ngathers(g + 2, h):
                        d.start()
            return carry

        lax.fori_loop(0, ngroups // 2, body, 0)

    return embed


def kernel(x, node_ids, W_word, W_graph):
    B, S = x.shape
    _, N = node_ids.shape
    return _make_embed(B, S, N)(x, node_ids, W_word, W_graph)


# speedup vs baseline: 1.0474x; 1.0076x over previous
import functools

import jax
import jax.numpy as jnp
from jax import lax
from jax.experimental import pallas as pl
from jax.experimental.pallas import tpu as pltpu
from jax.experimental.pallas import tpu_sc as plsc

D = 128
NW = 32
CW0 = 128
NBUF = 4
NGRP = 2


def _make_embed(B: int, S: int, N: int):
    BW = B * S
    BN = B * N
    RPW = B // NW
    wpw = RPW * S
    npw = RPW * N
    CW1 = S - CW0
    G = RPW // NGRP
    GID = NGRP * N
    assert RPW % NGRP == 0 and 0 < CW1 <= 128 and N <= 128

    mesh = plsc.VectorSubcoreMesh(core_axis_name="c", subcore_axis_name="s")

    @functools.partial(
        pl.kernel,
        mesh=mesh,
        out_type=(
            jax.ShapeDtypeStruct((B, S, D), jnp.float32),
            jax.ShapeDtypeStruct((B, N, D), jnp.float32),
        ),
        scratch_types=[
            pltpu.VMEM((RPW, S), jnp.int32),
            pltpu.VMEM((RPW, N), jnp.int32),
            pltpu.VMEM((NBUF // 2, CW0 + CW1, D), jnp.float32),
            pltpu.VMEM((2, NGRP, N, D), jnp.float32),
            pltpu.SemaphoreType.DMA,
            pltpu.SemaphoreType.DMA,
        ]
        + [pltpu.SemaphoreType.DMA] * 12,
    )
    def embed(x_hbm, nid_hbm, ww_hbm, wg_hbm, out_w, out_n,
              idxw_2d, idxn_2d, bufs_v, bufn_v, isem0, isem1, *sems):
        gsems = sems[0:4]
        osems = sems[4:8]
        nsems = sems[8:10]
        msems = sems[10:12]
        wid = lax.axis_index("s") * 2 + lax.axis_index("c")
        rbase = wid * RPW

        widx_cp = pltpu.make_async_copy(
            x_hbm.at[pl.ds(wid * RPW, RPW), :], idxw_2d, isem0)
        nidx_cp = pltpu.make_async_copy(
            nid_hbm.at[pl.ds(wid * RPW, RPW), :], idxn_2d, isem1)
        widx_cp.start()
        nidx_cp.start()

        def wsize(b):
            return CW0 if b % 2 == 0 else CW1

        def wbuf(b):
            return bufs_v.at[b // 2, pl.ds((b % 2) * CW0, wsize(b))]

        def wgather(j, b):
            row = j // 2
            idx = idxw_2d.at[row, pl.ds((b % 2) * CW0, wsize(b))]
            return pltpu.make_async_copy(ww_hbm.at[idx], wbuf(b), gsems[b])

        def wocopy(j, b):
            row = j // 2
            dst = out_w.at[rbase + row, pl.ds((b % 2) * CW0, wsize(b))]
            return pltpu.make_async_copy(wbuf(b), dst, osems[b])

        def ngathers(g, b):
            return [
                pltpu.make_async_copy(
                    wg_hbm.at[idxn_2d.at[g * NGRP + r]],
                    bufn_v.at[b, r], nsems[b])
                for r in range(NGRP)
            ]

        def nocopy(g, b):
            dst = out_n.at[pl.ds(rbase + g * NGRP, NGRP)]
            return pltpu.make_async_copy(bufn_v.at[b], dst, msems[b])

        nchunks = 2 * RPW
        ngroups = G

        widx_cp.wait()
        for b in range(4):
            wgather(b, b).start()
        nidx_cp.wait()
        for h in range(2):
            for d in ngathers(h, h):
                d.start()

        rounds = nchunks // (4 * ngroups)

        def body(i, carry):
            for h in range(2):
                g = 2 * i + h
                for q in range(rounds):
                    j0 = (g * rounds + q) * 4
                    for b in range(4):
                        wgather(j0 + b, b).wait()
                        wocopy(j0 + b, b).start()
                    if q == 0:
                        for d in ngathers(g, h):
                            d.wait()
                        nocopy(g, h).start()
                    for b in range(4):
                        wocopy(j0 + b, b).wait()

                        @pl.when(j0 + b + 4 < nchunks)
                        def _():
                            wgather(j0 + b + 4, b).start()
                    if q == rounds - 1:
                        nocopy(g, h).wait()

                        @pl.when(g + 2 < ngroups)
                        def _():
                            for d in ngathers(g + 2, h):
                                d.start()
            return carry

        lax.fori_loop(0, ngroups // 2, body, 0)

    return embed


def kernel(x, node_ids, W_word, W_graph):
    B, S = x.shape
    _, N = node_ids.shape
    return _make_embed(B, S, N)(x, node_ids, W_word, W_graph)
